# Initial kernel scaffold; baseline (speedup 1.0000x reference)
import functools

import jax
import jax.numpy as jnp
import numpy as np
from jax import lax
from jax.experimental import pallas as pl
from jax.experimental.pallas import tpu as pltpu
from jax.experimental.pallas import tpu_sc as plsc

N = 10000
E = 320000
D = 128
EMB = 16
H = 64

_z = np.linspace(-10.0, 10.0, 200001)
_pdf = np.exp(-0.5 * _z ** 2) / np.sqrt(2.0 * np.pi)
_a = np.logaddexp(0.0, _z) - np.log(2.0)
_SSP = float(1.0 / np.sqrt(np.trapz(_a ** 2 * _pdf, _z)))
_LOG2 = float(np.log(2.0))

_NC = 2
_NS = 16
_NW = _NC * _NS
CH = 128
NCHUNK = E // CH
ITERS = (NCHUNK + _NW - 1) // _NW
N_PAD = 10240
ROWS_PER_SUB = N_PAD // _NS


def _ssp(x):
    sp = jnp.maximum(x, 0.0) + jnp.log(1.0 + jnp.exp(-jnp.abs(x)))
    return (sp - _LOG2) * _SSP


def _coef_body(emb_ref, attr_ref, w1_ref, w2_ref, w3_ref, o_ref):
    h = _ssp(jnp.dot(emb_ref[...], w1_ref[...],
                     preferred_element_type=jnp.float32) * (1.0 / 4.0))
    h = _ssp(jnp.dot(h, w2_ref[...],
                     preferred_element_type=jnp.float32) * 0.125)
    w = jnp.dot(h, w3_ref[...], preferred_element_type=jnp.float32) * 0.125
    o_ref[...] = w * attr_ref[...] * (1.0 / 32.0)


def _edge_coefficients(edge_attr, edge_embedding, W1, W2, W3):
    blk = 3200
    grid = E // blk
    return pl.pallas_call(
        _coef_body,
        grid=(grid,),
        in_specs=[
            pl.BlockSpec((blk, EMB), lambda i: (i, 0)),
            pl.BlockSpec((blk, 1), lambda i: (i, 0)),
            pl.BlockSpec((EMB, H), lambda i: (0, 0)),
            pl.BlockSpec((H, H), lambda i: (0, 0)),
            pl.BlockSpec((H, D), lambda i: (0, 0)),
        ],
        out_specs=pl.BlockSpec((blk, D), lambda i: (i, 0)),
        out_shape=jax.ShapeDtypeStruct((E, D), jnp.float32),
    )(edge_embedding, edge_attr, W1, W2, W3)


def _sc_body(x_hbm, p_hbm, src_hbm, dst_hbm, out_hbm,
             src_v, dst_v, rows_v, p_v, acc_sh, sem):
    cid = lax.axis_index("c")
    sid = lax.axis_index("s")
    wid = sid * _NC + cid

    def _zrow(i, carry):
        for k in range(D // 16):
            rows_v[i, pl.ds(k * 16, 16)] = jnp.zeros((16,), jnp.float32)
        return carry
    lax.fori_loop(0, CH, _zrow, 0)
    for t in range(ROWS_PER_SUB // CH):
        pltpu.sync_copy(rows_v, acc_sh.at[pl.ds(sid * ROWS_PER_SUB + t * CH, CH)])
    plsc.subcore_barrier()

    def _chunk(j, carry):
        c = wid + j * _NW

        @pl.when(c < NCHUNK)
        def _():
            base = c * CH
            pltpu.sync_copy(src_hbm.at[pl.ds(base, CH)], src_v)
            pltpu.sync_copy(dst_hbm.at[pl.ds(base, CH)], dst_v)
            pltpu.async_copy(x_hbm.at[src_v], rows_v, sem).wait()
            pltpu.sync_copy(p_hbm.at[pl.ds(base, CH)], p_v)

            def _mul(i, c2):
                for k in range(D // 16):
                    s = pl.ds(k * 16, 16)
                    p_v[i, s] = p_v[i, s] * rows_v[i, s]
                return c2
            lax.fori_loop(0, CH, _mul, 0)
            pltpu.sync_copy(p_v, acc_sh.at[dst_v], add=True)
        return carry
    lax.fori_loop(0, ITERS, _chunk, 0)
    plsc.subcore_barrier()

    for t in range(ROWS_PER_SUB // CH):
        r = sid * ROWS_PER_SUB + t * CH
        pltpu.sync_copy(acc_sh.at[pl.ds(r, CH)], rows_v)
        pltpu.sync_copy(rows_v, out_hbm.at[cid, pl.ds(r, CH)])


def _scatter_partials(node_features, coef, edge_src, edge_dst):
    mesh = plsc.VectorSubcoreMesh(core_axis_name="c", subcore_axis_name="s")
    f = pl.kernel(
        _sc_body,
        out_type=jax.ShapeDtypeStruct((_NC, N_PAD, D), jnp.float32),
        mesh=mesh,
        scratch_types=[
            pltpu.VMEM((CH,), jnp.int32),
            pltpu.VMEM((CH,), jnp.int32),
            pltpu.VMEM((CH, D), jnp.float32),
            pltpu.VMEM((CH, D), jnp.float32),
            pltpu.VMEM_SHARED((N_PAD, D), jnp.float32),
            pltpu.SemaphoreType.DMA,
        ],
    )
    return f(node_features, coef, edge_src, edge_dst)


def _combine_body(p_ref, o_ref):
    o_ref[...] = p_ref[0] + p_ref[1]


def _combine(partials):
    blk = 500
    return pl.pallas_call(
        _combine_body,
        grid=(N // blk,),
        in_specs=[pl.BlockSpec((_NC, blk, D), lambda i: (0, i, 0))],
        out_specs=pl.BlockSpec((blk, D), lambda i: (i, 0)),
        out_shape=jax.ShapeDtypeStruct((N, D), jnp.float32),
    )(partials)


def kernel(node_features, edge_attr, edge_embedding, edge_index, W1, W2, W3):
    edge_src = edge_index[1]
    edge_dst = edge_index[0]
    coef = _edge_coefficients(edge_attr, edge_embedding, W1, W2, W3)
    partials = _scatter_partials(node_features, coef, edge_src, edge_dst)
    return _combine(partials)


# speedup vs baseline: 2.3857x; 2.3857x over previous
import functools

import jax
import jax.numpy as jnp
import numpy as np
from jax import lax
from jax.experimental import pallas as pl
from jax.experimental.pallas import tpu as pltpu
from jax.experimental.pallas import tpu_sc as plsc

N = 10000
E = 320000
D = 128
EMB = 16
H = 64

_z = np.linspace(-10.0, 10.0, 200001)
_pdf = np.exp(-0.5 * _z ** 2) / np.sqrt(2.0 * np.pi)
_a = np.logaddexp(0.0, _z) - np.log(2.0)
_SSP = float(1.0 / np.sqrt(np.trapz(_a ** 2 * _pdf, _z)))
_LOG2 = float(np.log(2.0))

_NC = 2
_NS = 16
_NW = _NC * _NS
CH = 128
NCHUNK = E // CH
ITERS = (NCHUNK + _NW - 1) // _NW
N_PAD = 10240
ROWS_PER_SUB = N_PAD // _NS


def _ssp(x):
    sp = jnp.maximum(x, 0.0) + jnp.log(1.0 + jnp.exp(-jnp.abs(x)))
    return (sp - _LOG2) * _SSP


def _coef_body(emb_ref, attr_ref, w1_ref, w2_ref, w3_ref, o_ref):
    h = _ssp(jnp.dot(emb_ref[...], w1_ref[...],
                     preferred_element_type=jnp.float32) * (1.0 / 4.0))
    h = _ssp(jnp.dot(h, w2_ref[...],
                     preferred_element_type=jnp.float32) * 0.125)
    w = jnp.dot(h, w3_ref[...], preferred_element_type=jnp.float32) * 0.125
    o_ref[...] = w * attr_ref[...] * (1.0 / 32.0)


def _edge_coefficients(edge_attr, edge_embedding, W1, W2, W3):
    blk = 3200
    grid = E // blk
    return pl.pallas_call(
        _coef_body,
        grid=(grid,),
        in_specs=[
            pl.BlockSpec((blk, EMB), lambda i: (i, 0)),
            pl.BlockSpec((blk, 1), lambda i: (i, 0)),
            pl.BlockSpec((EMB, H), lambda i: (0, 0)),
            pl.BlockSpec((H, H), lambda i: (0, 0)),
            pl.BlockSpec((H, D), lambda i: (0, 0)),
        ],
        out_specs=pl.BlockSpec((blk, D), lambda i: (i, 0)),
        out_shape=jax.ShapeDtypeStruct((E, D), jnp.float32),
    )(edge_embedding, edge_attr, W1, W2, W3)


def _sc_body(x_hbm, p_hbm, src_hbm, dst_hbm, out_hbm,
             src_v, dst_v, rows_v, p_v, acc_sh, sem):
    cid = lax.axis_index("c")
    sid = lax.axis_index("s")
    wid = sid * _NC + cid

    def _zrow(i, carry):
        for k in range(D // 16):
            rows_v[i, pl.ds(k * 16, 16)] = jnp.zeros((16,), jnp.float32)
        return carry
    lax.fori_loop(0, CH, _zrow, 0)
    for t in range(ROWS_PER_SUB // CH):
        pltpu.sync_copy(rows_v, acc_sh.at[pl.ds(sid * ROWS_PER_SUB + t * CH, CH)])
    plsc.subcore_barrier()

    def _chunk(j, carry):
        c = wid + j * _NW

        @pl.when(c < NCHUNK)
        def _():
            base = c * CH
            pltpu.sync_copy(src_hbm.at[pl.ds(base, CH)], src_v)
            pltpu.sync_copy(dst_hbm.at[pl.ds(base, CH)], dst_v)
            pltpu.async_copy(x_hbm.at[src_v], rows_v, sem).wait()
            pltpu.sync_copy(p_hbm.at[pl.ds(base, CH)], p_v)

            def _mul(i, c2):
                for k in range(D // 16):
                    s = pl.ds(k * 16, 16)
                    p_v[i, s] = p_v[i, s] * rows_v[i, s]
                return c2
            lax.fori_loop(0, CH, _mul, 0)
            pltpu.sync_copy(p_v, acc_sh.at[dst_v], add=True)
        return carry
    lax.fori_loop(0, ITERS, _chunk, 0)
    plsc.subcore_barrier()

    for t in range(ROWS_PER_SUB // CH):
        r = sid * ROWS_PER_SUB + t * CH
        pltpu.sync_copy(acc_sh.at[pl.ds(r, CH)], rows_v)
        pltpu.sync_copy(rows_v, out_hbm.at[cid, pl.ds(r, CH)])


def _scatter_partials(node_features, coef, edge_src, edge_dst):
    mesh = plsc.VectorSubcoreMesh(core_axis_name="c", subcore_axis_name="s")
    f = pl.kernel(
        _sc_body,
        out_type=jax.ShapeDtypeStruct((_NC, N_PAD, D), jnp.float32),
        mesh=mesh,
        scratch_types=[
            pltpu.VMEM((CH,), jnp.int32),
            pltpu.VMEM((CH,), jnp.int32),
            pltpu.VMEM((CH, D), jnp.float32),
            pltpu.VMEM((CH, D), jnp.float32),
            pltpu.VMEM_SHARED((N_PAD, D), jnp.float32),
            pltpu.SemaphoreType.DMA,
        ],
    )
    return f(node_features, coef, edge_src, edge_dst)


def _combine_body(p_ref, o_ref):
    o_ref[...] = p_ref[0] + p_ref[1]


def _combine(partials):
    blk = 1000
    return pl.pallas_call(
        _combine_body,
        grid=(N // blk,),
        in_specs=[pl.BlockSpec((_NC, blk, D), lambda i: (0, i, 0))],
        out_specs=pl.BlockSpec((blk, D), lambda i: (i, 0)),
        out_shape=jax.ShapeDtypeStruct((N, D), jnp.float32),
    )(partials)


def kernel(node_features, edge_attr, edge_embedding, edge_index, W1, W2, W3):
    edge_src = edge_index[1]
    edge_dst = edge_index[0]
    coef = _edge_coefficients(edge_attr, edge_embedding, W1, W2, W3)
    partials = _scatter_partials(node_features, coef, edge_src, edge_dst)
    return _combine(partials)


# speedup vs baseline: 2.9967x; 1.2561x over previous
import functools

import jax
import jax.numpy as jnp
import numpy as np
from jax import lax
from jax.experimental import pallas as pl
from jax.experimental.pallas import tpu as pltpu
from jax.experimental.pallas import tpu_sc as plsc

N = 10000
E = 320000
D = 128
EMB = 16
H = 64

_z = np.linspace(-10.0, 10.0, 200001)
_pdf = np.exp(-0.5 * _z ** 2) / np.sqrt(2.0 * np.pi)
_a = np.logaddexp(0.0, _z) - np.log(2.0)
_SSP = float(1.0 / np.sqrt(np.trapz(_a ** 2 * _pdf, _z)))
_LOG2 = float(np.log(2.0))

_NC = 2
_NS = 16
_NW = _NC * _NS
CH = 128
NCHUNK = E // CH
ITERS = (NCHUNK + _NW - 1) // _NW
N_PAD = 10240
ROWS_PER_SUB = N_PAD // _NS


def _ssp(x):
    sp = jnp.maximum(x, 0.0) + jnp.log(1.0 + jnp.exp(-jnp.abs(x)))
    return (sp - _LOG2) * _SSP


def _coef_body(embt_ref, attr_ref, w1_ref, w2_ref, w3_ref, o_ref):
    h = lax.dot_general(embt_ref[...], w1_ref[...],
                        (((0,), (0,)), ((), ())),
                        preferred_element_type=jnp.float32) * (1.0 / 4.0)
    h = _ssp(h)
    h = _ssp(jnp.dot(h, w2_ref[...],
                     preferred_element_type=jnp.float32) * 0.125)
    w = jnp.dot(h, w3_ref[...], preferred_element_type=jnp.float32) * 0.125
    a = jnp.transpose(attr_ref[...])
    o_ref[...] = w * a * (1.0 / 32.0)


def _edge_coefficients(edge_embedding_t, edge_attr_t, W1, W2, W3):
    blk = 3200
    grid = E // blk
    return pl.pallas_call(
        _coef_body,
        grid=(grid,),
        in_specs=[
            pl.BlockSpec((EMB, blk), lambda i: (0, i)),
            pl.BlockSpec((1, blk), lambda i: (0, i)),
            pl.BlockSpec((EMB, H), lambda i: (0, 0)),
            pl.BlockSpec((H, H), lambda i: (0, 0)),
            pl.BlockSpec((H, D), lambda i: (0, 0)),
        ],
        out_specs=pl.BlockSpec((blk, D), lambda i: (i, 0)),
        out_shape=jax.ShapeDtypeStruct((E, D), jnp.float32),
    )(edge_embedding_t, edge_attr_t, W1, W2, W3)


def _sc_body(x_hbm, p_hbm, src_hbm, dst_hbm, out_hbm,
             src_v, dst_v, rows_v, p_v, acc_sh, sem):
    cid = lax.axis_index("c")
    sid = lax.axis_index("s")
    wid = sid * _NC + cid

    def _zrow(i, carry):
        for k in range(D // 16):
            rows_v[i, pl.ds(k * 16, 16)] = jnp.zeros((16,), jnp.float32)
        return carry
    lax.fori_loop(0, CH, _zrow, 0)
    for t in range(ROWS_PER_SUB // CH):
        pltpu.sync_copy(rows_v, acc_sh.at[pl.ds(sid * ROWS_PER_SUB + t * CH, CH)])
    plsc.subcore_barrier()

    def _chunk(j, carry):
        c = wid + j * _NW

        @pl.when(c < NCHUNK)
        def _():
            base = c * CH
            pltpu.sync_copy(src_hbm.at[pl.ds(base, CH)], src_v)
            pltpu.sync_copy(dst_hbm.at[pl.ds(base, CH)], dst_v)
            pltpu.async_copy(x_hbm.at[src_v], rows_v, sem).wait()
            pltpu.sync_copy(p_hbm.at[pl.ds(base, CH)], p_v)

            def _mul(i, c2):
                for k in range(D // 16):
                    s = pl.ds(k * 16, 16)
                    p_v[i, s] = p_v[i, s] * rows_v[i, s]
                return c2
            lax.fori_loop(0, CH, _mul, 0)
            pltpu.sync_copy(p_v, acc_sh.at[dst_v], add=True)
        return carry
    lax.fori_loop(0, ITERS, _chunk, 0)
    plsc.subcore_barrier()

    for t in range(ROWS_PER_SUB // CH):
        r = sid * ROWS_PER_SUB + t * CH
        pltpu.sync_copy(acc_sh.at[pl.ds(r, CH)], rows_v)
        pltpu.sync_copy(rows_v, out_hbm.at[cid, pl.ds(r, CH)])


def _scatter_partials(node_features, coef, edge_src, edge_dst):
    mesh = plsc.VectorSubcoreMesh(core_axis_name="c", subcore_axis_name="s")
    f = pl.kernel(
        _sc_body,
        out_type=jax.ShapeDtypeStruct((_NC, N_PAD, D), jnp.float32),
        mesh=mesh,
        scratch_types=[
            pltpu.VMEM((CH,), jnp.int32),
            pltpu.VMEM((CH,), jnp.int32),
            pltpu.VMEM((CH, D), jnp.float32),
            pltpu.VMEM((CH, D), jnp.float32),
            pltpu.VMEM_SHARED((N_PAD, D), jnp.float32),
            pltpu.SemaphoreType.DMA,
        ],
    )
    return f(node_features, coef, edge_src, edge_dst)


def _combine_body(p_ref, o_ref):
    o_ref[...] = p_ref[0] + p_ref[1]


def _combine(partials):
    blk = 1000
    return pl.pallas_call(
        _combine_body,
        grid=(N // blk,),
        in_specs=[pl.BlockSpec((_NC, blk, D), lambda i: (0, i, 0))],
        out_specs=pl.BlockSpec((blk, D), lambda i: (i, 0)),
        out_shape=jax.ShapeDtypeStruct((N, D), jnp.float32),
    )(partials)


def kernel(node_features, edge_attr, edge_embedding, edge_index, W1, W2, W3):
    edge_src = edge_index[1]
    edge_dst = edge_index[0]
    coef = _edge_coefficients(edge_embedding.T, edge_attr.T, W1, W2, W3)
    partials = _scatter_partials(node_features, coef, edge_src, edge_dst)
    return _combine(partials)


# speedup vs baseline: 4.7333x; 1.5795x over previous
import functools

import jax
import jax.numpy as jnp
import numpy as np
from jax import lax
from jax.experimental import pallas as pl
from jax.experimental.pallas import tpu as pltpu
from jax.experimental.pallas import tpu_sc as plsc

N = 10000
E = 320000
D = 128
EMB = 16
H = 64

_z = np.linspace(-10.0, 10.0, 200001)
_pdf = np.exp(-0.5 * _z ** 2) / np.sqrt(2.0 * np.pi)
_a = np.logaddexp(0.0, _z) - np.log(2.0)
_SSP = float(1.0 / np.sqrt(np.trapz(_a ** 2 * _pdf, _z)))
_LOG2 = float(np.log(2.0))

_NC = 2
_NS = 16
_NW = _NC * _NS
CH = 80
NCHUNK = E // CH
ITERS = NCHUNK // _NW
N_PAD = 10240
ROWS_PER_SUB = N_PAD // _NS


def _ssp(x):
    sp = jnp.maximum(x, 0.0) + jnp.log(1.0 + jnp.exp(-jnp.abs(x)))
    return (sp - _LOG2) * _SSP


def _coef_body(embt_ref, attr_ref, w1_ref, w2_ref, w3_ref, o_ref):
    h = lax.dot_general(embt_ref[...], w1_ref[...],
                        (((0,), (0,)), ((), ())),
                        preferred_element_type=jnp.float32) * (1.0 / 4.0)
    h = _ssp(h)
    h = _ssp(jnp.dot(h, w2_ref[...],
                     preferred_element_type=jnp.float32) * 0.125)
    w = jnp.dot(h, w3_ref[...], preferred_element_type=jnp.float32) * 0.125
    a = jnp.transpose(attr_ref[...])
    o_ref[...] = w * a * (1.0 / 32.0)


def _edge_coefficients(edge_embedding_t, edge_attr_t, W1, W2, W3):
    blk = 3200
    grid = E // blk
    return pl.pallas_call(
        _coef_body,
        grid=(grid,),
        in_specs=[
            pl.BlockSpec((EMB, blk), lambda i: (0, i)),
            pl.BlockSpec((1, blk), lambda i: (0, i)),
            pl.BlockSpec((EMB, H), lambda i: (0, 0)),
            pl.BlockSpec((H, H), lambda i: (0, 0)),
            pl.BlockSpec((H, D), lambda i: (0, 0)),
        ],
        out_specs=pl.BlockSpec((blk, D), lambda i: (i, 0)),
        out_shape=jax.ShapeDtypeStruct((E, D), jnp.float32),
    )(edge_embedding_t, edge_attr_t, W1, W2, W3)


def _sc_body(x_hbm, p_hbm, src_hbm, dst_hbm, out_hbm,
             src_v, dst_v, sdst_v, rows_v, p_v, acc_sh,
             s_src, s_dst, s_g, s_p):
    cid = lax.axis_index("c")
    sid = lax.axis_index("s")
    wid = sid * _NC + cid

    def _base(j):
        return (wid + j * _NW) * CH

    def _valid(j):
        return (wid + j * _NW) < NCHUNK

    def _issue_idx(j, b):
        pltpu.async_copy(src_hbm.at[pl.ds(_base(j), CH)], src_v.at[b], s_src.at[b])
        pltpu.async_copy(dst_hbm.at[pl.ds(_base(j), CH)], dst_v.at[b], s_dst.at[b])

    def _wait_idx(j, b):
        pltpu.make_async_copy(src_hbm.at[pl.ds(_base(j), CH)], src_v.at[b], s_src.at[b]).wait()
        pltpu.make_async_copy(dst_hbm.at[pl.ds(_base(j), CH)], dst_v.at[b], s_dst.at[b]).wait()

    def _issue_data(j, b):
        pltpu.async_copy(x_hbm.at[src_v.at[b]], rows_v.at[b], s_g.at[b])
        pltpu.async_copy(p_hbm.at[pl.ds(_base(j), CH)], p_v.at[b], s_p.at[b])

    def _wait_data(j, b):
        pltpu.make_async_copy(x_hbm.at[src_v.at[b]], rows_v.at[b], s_g.at[b]).wait()
        pltpu.make_async_copy(p_hbm.at[pl.ds(_base(j), CH)], p_v.at[b], s_p.at[b]).wait()

    def _zrow(i, carry):
        for k in range(D // 16):
            rows_v[0, i, pl.ds(k * 16, 16)] = jnp.zeros((16,), jnp.float32)
        return carry
    lax.fori_loop(0, CH, _zrow, 0)
    for t in range(ROWS_PER_SUB // CH):
        pltpu.sync_copy(rows_v.at[0],
                        acc_sh.at[pl.ds(sid * ROWS_PER_SUB + t * CH, CH)])
    plsc.subcore_barrier()

    pltpu.sync_copy(src_hbm.at[pl.ds(_base(0), CH)], src_v.at[0])
    pltpu.sync_copy(dst_hbm.at[pl.ds(_base(0), CH)], dst_v.at[0])
    _issue_data(0, 0)
    _issue_idx(1, 1)

    def _outer(g, carry):
        for b in (0, 1):
            j = g * 2 + b
            b2 = 1 - b

            @pl.when(_valid(j + 1))
            def _():
                _wait_idx(j + 1, b2)
                _issue_data(j + 1, b2)

            @pl.when(_valid(j))
            def _():
                _wait_data(j, b)
                for k in range(CH // 16):
                    s = pl.ds(k * 16, 16)
                    sdst_v[b, s] = dst_v[b, s]

                @pl.when(_valid(j + 2))
                def _():
                    _issue_idx(j + 2, b)

                def _mul(i, c2):
                    for k in range(D // 16):
                        s = pl.ds(k * 16, 16)
                        p_v[b, i, s] = p_v[b, i, s] * rows_v[b, i, s]
                    return c2
                lax.fori_loop(0, CH, _mul, 0)
                pltpu.sync_copy(p_v.at[b], acc_sh.at[sdst_v.at[b]], add=True)
        return carry
    lax.fori_loop(0, (ITERS + 1) // 2, _outer, 0)
    plsc.subcore_barrier()

    for t in range(ROWS_PER_SUB // CH):
        r = sid * ROWS_PER_SUB + t * CH
        pltpu.sync_copy(acc_sh.at[pl.ds(r, CH)], rows_v.at[0])
        pltpu.sync_copy(rows_v.at[0], out_hbm.at[cid, pl.ds(r, CH)])


def _scatter_partials(node_features, coef, edge_src, edge_dst):
    mesh = plsc.VectorSubcoreMesh(core_axis_name="c", subcore_axis_name="s")
    f = pl.kernel(
        _sc_body,
        out_type=jax.ShapeDtypeStruct((_NC, N_PAD, D), jnp.float32),
        mesh=mesh,
        scratch_types=[
            pltpu.VMEM((2, CH), jnp.int32),
            pltpu.VMEM((2, CH), jnp.int32),
            pltpu.VMEM((2, CH), jnp.int32),
            pltpu.VMEM((2, CH, D), jnp.float32),
            pltpu.VMEM((2, CH, D), jnp.float32),
            pltpu.VMEM_SHARED((N_PAD, D), jnp.float32),
            pltpu.SemaphoreType.DMA((2,)),
            pltpu.SemaphoreType.DMA((2,)),
            pltpu.SemaphoreType.DMA((2,)),
            pltpu.SemaphoreType.DMA((2,)),
        ],
    )
    return f(node_features, coef, edge_src, edge_dst)


def _combine_body(p_ref, o_ref):
    o_ref[...] = p_ref[0] + p_ref[1]


def _combine(partials):
    blk = 1000
    return pl.pallas_call(
        _combine_body,
        grid=(N // blk,),
        in_specs=[pl.BlockSpec((_NC, blk, D), lambda i: (0, i, 0))],
        out_specs=pl.BlockSpec((blk, D), lambda i: (i, 0)),
        out_shape=jax.ShapeDtypeStruct((N, D), jnp.float32),
    )(partials)


def kernel(node_features, edge_attr, edge_embedding, edge_index, W1, W2, W3):
    edge_src = edge_index[1]
    edge_dst = edge_index[0]
    coef = _edge_coefficients(edge_embedding.T, edge_attr.T, W1, W2, W3)
    partials = _scatter_partials(node_features, coef, edge_src, edge_dst)
    return _combine(partials)


# speedup vs baseline: 5.8680x; 1.2397x over previous
import jax
import jax.numpy as jnp
import numpy as np
from jax import lax
from jax.experimental import pallas as pl
from jax.experimental.pallas import tpu as pltpu
from jax.experimental.pallas import tpu_sc as plsc

N = 10000
E = 320000
D = 128
EMB = 16
H = 64

_z = np.linspace(-10.0, 10.0, 200001)
_pdf = np.exp(-0.5 * _z ** 2) / np.sqrt(2.0 * np.pi)
_a = np.logaddexp(0.0, _z) - np.log(2.0)
_SSP = float(1.0 / np.sqrt(np.trapz(_a ** 2 * _pdf, _z)))
_LOG2 = float(np.log(2.0))

G = 2
EG = E // G

_NC = 2
_NS = 16
_NW = _NC * _NS
CH = 80
NCHUNK = EG // CH
ITERS = (NCHUNK + _NW - 1) // _NW
N_PAD = 10240
ROWS_PER_SUB = N_PAD // _NS

BLK = 3200
BPG = EG // BLK


def _sp(x):
    return jnp.maximum(x, 0.0) + jnp.log(1.0 + jnp.exp(-jnp.abs(x)))


def _coef_body(embt_ref, attr_ref, w1_ref, w2_ref, b2_ref, w3_ref, b3_ref,
               o_ref):
    s1 = _sp(lax.dot_general(embt_ref[...], w1_ref[...],
                             (((0,), (0,)), ((), ())),
                             preferred_element_type=jnp.float32))
    s2 = _sp(jnp.dot(s1, w2_ref[...],
                     preferred_element_type=jnp.float32) + b2_ref[...])
    w = jnp.dot(s2, w3_ref[...], preferred_element_type=jnp.float32) + b3_ref[...]
    a = jnp.transpose(attr_ref[...])
    o_ref[...] = w * a


def _edge_coefficients(g, edge_embedding_t, edge_attr_t, W1a, W2a, b2, W3a, b3):
    return pl.pallas_call(
        _coef_body,
        grid=(BPG,),
        in_specs=[
            pl.BlockSpec((EMB, BLK), lambda i: (0, i + g * BPG)),
            pl.BlockSpec((1, BLK), lambda i: (0, i + g * BPG)),
            pl.BlockSpec((EMB, H), lambda i: (0, 0)),
            pl.BlockSpec((H, H), lambda i: (0, 0)),
            pl.BlockSpec((1, H), lambda i: (0, 0)),
            pl.BlockSpec((H, D), lambda i: (0, 0)),
            pl.BlockSpec((1, D), lambda i: (0, 0)),
        ],
        out_specs=pl.BlockSpec((BLK, D), lambda i: (i, 0)),
        out_shape=jax.ShapeDtypeStruct((EG, D), jnp.float32),
    )(edge_embedding_t, edge_attr_t, W1a, W2a, b2, W3a, b3)


def _sc_body(x_hbm, p_hbm, src_hbm, dst_hbm, out_hbm,
             src_v, dst_v, sdst_v, rows_v, p_v, acc_sh,
             s_src, s_dst, s_g, s_p):
    cid = lax.axis_index("c")
    sid = lax.axis_index("s")
    wid = sid * _NC + cid

    def _base(j):
        return (wid + j * _NW) * CH

    def _valid(j):
        return (wid + j * _NW) < NCHUNK

    def _issue_idx(j, b):
        pltpu.async_copy(src_hbm.at[pl.ds(_base(j), CH)], src_v.at[b], s_src.at[b])
        pltpu.async_copy(dst_hbm.at[pl.ds(_base(j), CH)], dst_v.at[b], s_dst.at[b])

    def _wait_idx(j, b):
        pltpu.make_async_copy(src_hbm.at[pl.ds(_base(j), CH)], src_v.at[b], s_src.at[b]).wait()
        pltpu.make_async_copy(dst_hbm.at[pl.ds(_base(j), CH)], dst_v.at[b], s_dst.at[b]).wait()

    def _issue_data(j, b):
        pltpu.async_copy(x_hbm.at[src_v.at[b]], rows_v.at[b], s_g.at[b])
        pltpu.async_copy(p_hbm.at[pl.ds(_base(j), CH)], p_v.at[b], s_p.at[b])

    def _wait_data(j, b):
        pltpu.make_async_copy(x_hbm.at[src_v.at[b]], rows_v.at[b], s_g.at[b]).wait()
        pltpu.make_async_copy(p_hbm.at[pl.ds(_base(j), CH)], p_v.at[b], s_p.at[b]).wait()

    def _zrow(i, carry):
        for k in range(D // 16):
            rows_v[0, i, pl.ds(k * 16, 16)] = jnp.zeros((16,), jnp.float32)
        return carry
    lax.fori_loop(0, CH, _zrow, 0)
    for t in range(ROWS_PER_SUB // CH):
        pltpu.sync_copy(rows_v.at[0],
                        acc_sh.at[pl.ds(sid * ROWS_PER_SUB + t * CH, CH)])
    plsc.subcore_barrier()

    pltpu.sync_copy(src_hbm.at[pl.ds(_base(0), CH)], src_v.at[0])
    pltpu.sync_copy(dst_hbm.at[pl.ds(_base(0), CH)], dst_v.at[0])
    _issue_data(0, 0)
    _issue_idx(1, 1)

    def _outer(g, carry):
        for b in (0, 1):
            j = g * 2 + b
            b2 = 1 - b

            @pl.when(_valid(j + 1))
            def _():
                _wait_idx(j + 1, b2)
                _issue_data(j + 1, b2)

            @pl.when(_valid(j))
            def _():
                _wait_data(j, b)
                for k in range(CH // 16):
                    s = pl.ds(k * 16, 16)
                    sdst_v[b, s] = dst_v[b, s]

                @pl.when(_valid(j + 2))
                def _():
                    _issue_idx(j + 2, b)

                def _mul(i, c2):
                    for k in range(D // 16):
                        s = pl.ds(k * 16, 16)
                        p_v[b, i, s] = p_v[b, i, s] * rows_v[b, i, s]
                    return c2
                lax.fori_loop(0, CH, _mul, 0)
                pltpu.sync_copy(p_v.at[b], acc_sh.at[sdst_v.at[b]], add=True)
        return carry
    lax.fori_loop(0, (ITERS + 1) // 2, _outer, 0)
    plsc.subcore_barrier()

    for t in range(ROWS_PER_SUB // CH):
        r = sid * ROWS_PER_SUB + t * CH
        pltpu.sync_copy(acc_sh.at[pl.ds(r, CH)], rows_v.at[0])
        pltpu.sync_copy(rows_v.at[0], out_hbm.at[cid, pl.ds(r, CH)])


def _scatter_partials(node_features, coef, edge_src, edge_dst):
    mesh = plsc.VectorSubcoreMesh(core_axis_name="c", subcore_axis_name="s")
    f = pl.kernel(
        _sc_body,
        out_type=jax.ShapeDtypeStruct((_NC, N_PAD, D), jnp.float32),
        mesh=mesh,
        scratch_types=[
            pltpu.VMEM((2, CH), jnp.int32),
            pltpu.VMEM((2, CH), jnp.int32),
            pltpu.VMEM((2, CH), jnp.int32),
            pltpu.VMEM((2, CH, D), jnp.float32),
            pltpu.VMEM((2, CH, D), jnp.float32),
            pltpu.VMEM_SHARED((N_PAD, D), jnp.float32),
            pltpu.SemaphoreType.DMA((2,)),
            pltpu.SemaphoreType.DMA((2,)),
            pltpu.SemaphoreType.DMA((2,)),
            pltpu.SemaphoreType.DMA((2,)),
        ],
    )
    return f(node_features, coef, edge_src, edge_dst)


def _combine_body(p0_ref, p1_ref, o_ref):
    o_ref[...] = (p0_ref[0] + p0_ref[1]) + (p1_ref[0] + p1_ref[1])


def _combine(part0, part1):
    blk = 1000
    return pl.pallas_call(
        _combine_body,
        grid=(N // blk,),
        in_specs=[
            pl.BlockSpec((_NC, blk, D), lambda i: (0, i, 0)),
            pl.BlockSpec((_NC, blk, D), lambda i: (0, i, 0)),
        ],
        out_specs=pl.BlockSpec((blk, D), lambda i: (i, 0)),
        out_shape=jax.ShapeDtypeStruct((N, D), jnp.float32),
    )(part0, part1)


def kernel(node_features, edge_attr, edge_embedding, edge_index, W1, W2, W3):
    W1a = W1 * 0.25
    W2a = W2 * (_SSP * 0.125)
    b2 = (-_LOG2) * jnp.sum(W2a, axis=0, keepdims=True)
    W3a = W3 * (_SSP * (0.125 / 32.0))
    b3 = (-_LOG2) * jnp.sum(W3a, axis=0, keepdims=True)

    embt = edge_embedding.T
    attrt = edge_attr.T
    parts = []
    for g in range(G):
        coef = _edge_coefficients(g, embt, attrt, W1a, W2a, b2, W3a, b3)
        src = lax.slice(edge_index[1], (g * EG,), ((g + 1) * EG,))
        dst = lax.slice(edge_index[0], (g * EG,), ((g + 1) * EG,))
        parts.append(_scatter_partials(node_features, coef, src, dst))
    return _combine(parts[0], parts[1])
